# baseline (speedup 1.0000x reference)
import math

import jax
import jax.numpy as jnp
from jax import lax
from jax.experimental import pallas as pl
from jax.experimental.pallas import tpu as pltpu
from jax.experimental.pallas import tpu_sc as plsc

T = 32768
D_FEAT = 32
K = 16
NW = 32
CHUNK = T // NW
DB = 16


def _sc_body(x_hbm, mu_t_hbm, ls_t_hbm, out_hbm, x_v, out_v, s_v, m_v):
    wid = lax.axis_index("s") * 2 + lax.axis_index("c")
    base = wid * CHUNK

    pltpu.sync_copy(x_hbm.at[pl.ds(base * D_FEAT, CHUNK * D_FEAT)], x_v)
    pltpu.sync_copy(mu_t_hbm, m_v)
    pltpu.sync_copy(ls_t_hbm, s_v)

    c = jnp.full((K,), -0.5 * D_FEAT * math.log(2.0 * math.pi), jnp.float32)
    for d in range(D_FEAT):
        mu = m_v[pl.ds(d * K, K)]
        ls = s_v[pl.ds(d * K, K)]
        s = jnp.exp(-ls)
        c = c - ls
        s_v[pl.ds(d * K, K)] = s
        m_v[pl.ds(d * K, K)] = mu * s

    num_blocks = D_FEAT // DB
    for db in range(num_blocks):
        s_w = [s_v[pl.ds((db * DB + j) * K, K)] for j in range(DB)]
        m_w = [m_v[pl.ds((db * DB + j) * K, K)] for j in range(DB)]
        half = (db * DB // 16) * 16
        off = db * DB - half

        @plsc.parallel_loop(0, CHUNK, 1, unroll=4)
        def _token_body(t, _db=db, _s=s_w, _m=m_w, _half=half, _off=off):
            xv = x_v[pl.ds(t * D_FEAT + _half, 16)]
            if _db == 0:
                acc = jnp.zeros((K,), jnp.float32)
            else:
                acc = out_v[pl.ds(t * K, K)]
            for j in range(DB):
                z = xv[_off + j] * _s[j] - _m[j]
                acc = acc + z * z
            if _db == num_blocks - 1:
                acc = c - 0.5 * acc
            out_v[pl.ds(t * K, K)] = acc

    pltpu.sync_copy(out_v, out_hbm.at[pl.ds(base * K, CHUNK * K)])


@jax.jit
def _emission_log_probs(X, means, log_scales):
    mesh = plsc.VectorSubcoreMesh(core_axis_name="c", subcore_axis_name="s")
    run = pl.kernel(
        _sc_body,
        out_type=jax.ShapeDtypeStruct((T * K,), jnp.float32),
        mesh=mesh,
        scratch_types=[
            pltpu.VMEM((CHUNK * D_FEAT,), jnp.float32),
            pltpu.VMEM((CHUNK * K,), jnp.float32),
            pltpu.VMEM((D_FEAT * K,), jnp.float32),
            pltpu.VMEM((D_FEAT * K,), jnp.float32),
        ],
    )
    out = run(X.reshape(-1), means.T.reshape(-1), log_scales.T.reshape(-1))
    return out.reshape(T, K)


def kernel(X, cu_seqlens, means, log_scales, pi_logits, A_logits, D_logits):
    return _emission_log_probs(
        X.astype(jnp.float32),
        means.astype(jnp.float32),
        log_scales.astype(jnp.float32),
    )


# speedup vs baseline: 1.7761x; 1.7761x over previous
import math

import jax
import jax.numpy as jnp
from jax import lax
from jax.experimental import pallas as pl
from jax.experimental.pallas import tpu as pltpu
from jax.experimental.pallas import tpu_sc as plsc

T = 32768
D_FEAT = 32
K = 16
NW = 32
T_SC = 8192
CHUNK = T_SC // NW
DB = 16
TC_BLOCK = 4096
LOG2PI = math.log(2.0 * math.pi)


def _sc_body(x_hbm, mu_t_hbm, ls_t_hbm, out_hbm, x_v, out_v, s_v, m_v):
    wid = lax.axis_index("s") * 2 + lax.axis_index("c")
    base = wid * CHUNK

    pltpu.sync_copy(x_hbm.at[pl.ds(base * D_FEAT, CHUNK * D_FEAT)], x_v)
    pltpu.sync_copy(mu_t_hbm, m_v)
    pltpu.sync_copy(ls_t_hbm, s_v)

    c = jnp.full((K,), -0.5 * D_FEAT * LOG2PI, jnp.float32)
    for d in range(D_FEAT):
        mu = m_v[pl.ds(d * K, K)]
        ls = s_v[pl.ds(d * K, K)]
        s = jnp.exp(-ls)
        c = c - ls
        s_v[pl.ds(d * K, K)] = s
        m_v[pl.ds(d * K, K)] = mu * s

    dn = lax.GatherDimensionNumbers(
        offset_dims=(), collapsed_slice_dims=(0,), start_index_map=(0,))
    idxs = [jnp.full((16, 1), j, jnp.int32) for j in range(16)]

    def _bcast(vec, j):
        return lax.gather(vec, idxs[j], dn, slice_sizes=(1,),
                          mode=lax.GatherScatterMode.PROMISE_IN_BOUNDS)

    num_blocks = D_FEAT // DB
    for db in range(num_blocks):
        s_w = [s_v[pl.ds((db * DB + j) * K, K)] for j in range(DB)]
        m_w = [m_v[pl.ds((db * DB + j) * K, K)] for j in range(DB)]
        half = (db * DB // 16) * 16
        off = db * DB - half

        @plsc.parallel_loop(0, CHUNK, 1, unroll=2)
        def _token_body(t, _db=db, _s=s_w, _m=m_w, _half=half, _off=off):
            xv = x_v[pl.ds(t * D_FEAT + _half, 16)]
            if _db == 0:
                acc_a = jnp.zeros((K,), jnp.float32)
            else:
                acc_a = out_v[pl.ds(t * K, K)]
            acc_b = jnp.zeros((K,), jnp.float32)
            for j in range(0, DB, 2):
                z0 = _bcast(xv, _off + j) * _s[j] - _m[j]
                acc_a = acc_a + z0 * z0
                z1 = _bcast(xv, _off + j + 1) * _s[j + 1] - _m[j + 1]
                acc_b = acc_b + z1 * z1
            acc = acc_a + acc_b
            if _db == num_blocks - 1:
                acc = c - 0.5 * acc
            out_v[pl.ds(t * K, K)] = acc

    pltpu.sync_copy(out_v, out_hbm.at[pl.ds(base * K, CHUNK * K)])


def _sc_log_probs(X_sc, means, log_scales):
    mesh = plsc.VectorSubcoreMesh(core_axis_name="c", subcore_axis_name="s")
    run = pl.kernel(
        _sc_body,
        out_type=jax.ShapeDtypeStruct((T_SC * K,), jnp.float32),
        mesh=mesh,
        scratch_types=[
            pltpu.VMEM((CHUNK * D_FEAT,), jnp.float32),
            pltpu.VMEM((CHUNK * K,), jnp.float32),
            pltpu.VMEM((D_FEAT * K,), jnp.float32),
            pltpu.VMEM((D_FEAT * K,), jnp.float32),
        ],
    )
    out = run(X_sc.reshape(-1), means.T.reshape(-1), log_scales.T.reshape(-1))
    return out.reshape(T_SC, K)


def _tc_body(x_ref, mu_ref, ls_ref, o_ref):
    mu = mu_ref[...]
    ls = ls_ref[...]
    iv = jnp.exp(-2.0 * ls)
    w = jnp.concatenate([mu * iv, -0.5 * iv], axis=1)
    c = (jnp.sum(-0.5 * mu * mu * iv - ls, axis=1)
         - 0.5 * D_FEAT * LOG2PI)
    x = x_ref[...]
    xx = jnp.concatenate([x, x * x], axis=1)
    o_ref[...] = jax.lax.dot_general(
        xx, w, (((1,), (1,)), ((), ())),
        preferred_element_type=jnp.float32) + c[None, :]


def _tc_log_probs(X_tc, means, log_scales):
    t_tc = X_tc.shape[0]
    return pl.pallas_call(
        _tc_body,
        grid=(t_tc // TC_BLOCK,),
        in_specs=[
            pl.BlockSpec((TC_BLOCK, D_FEAT), lambda i: (i, 0)),
            pl.BlockSpec((K, D_FEAT), lambda i: (0, 0)),
            pl.BlockSpec((K, D_FEAT), lambda i: (0, 0)),
        ],
        out_specs=pl.BlockSpec((TC_BLOCK, K), lambda i: (i, 0)),
        out_shape=jax.ShapeDtypeStruct((t_tc, K), jnp.float32),
    )(X_tc, means, log_scales)


@jax.jit
def _emission_log_probs(X, means, log_scales):
    out_sc = _sc_log_probs(X[:T_SC], means, log_scales)
    out_tc = _tc_log_probs(X[T_SC:], means, log_scales)
    return jnp.concatenate([out_sc, out_tc], axis=0)


def kernel(X, cu_seqlens, means, log_scales, pi_logits, A_logits, D_logits):
    return _emission_log_probs(
        X.astype(jnp.float32),
        means.astype(jnp.float32),
        log_scales.astype(jnp.float32),
    )


# speedup vs baseline: 1.9044x; 1.0722x over previous
import math

import jax
import jax.numpy as jnp
from jax import lax
from jax.experimental import pallas as pl
from jax.experimental.pallas import tpu as pltpu
from jax.experimental.pallas import tpu_sc as plsc

T = 32768
D_FEAT = 32
K = 16
NW = 16
T_SC = 2048
CHUNK = T_SC // NW
DB = 16
TC_BLOCK = 6144
LOG2PI = math.log(2.0 * math.pi)


def _sc_body(x_hbm, mu_t_hbm, ls_t_hbm, out_hbm, x_v, out_v, s_v, m_v):
    wid = lax.axis_index("s")
    base = wid * CHUNK

    pltpu.sync_copy(x_hbm.at[pl.ds(base * D_FEAT, CHUNK * D_FEAT)], x_v)
    pltpu.sync_copy(mu_t_hbm, m_v)
    pltpu.sync_copy(ls_t_hbm, s_v)

    c = jnp.full((K,), -0.5 * D_FEAT * LOG2PI, jnp.float32)
    for d in range(D_FEAT):
        mu = m_v[pl.ds(d * K, K)]
        ls = s_v[pl.ds(d * K, K)]
        s = jnp.exp(-ls)
        c = c - ls
        s_v[pl.ds(d * K, K)] = s
        m_v[pl.ds(d * K, K)] = mu * s

    dn = lax.GatherDimensionNumbers(
        offset_dims=(), collapsed_slice_dims=(0,), start_index_map=(0,))
    idxs = [jnp.full((16, 1), j, jnp.int32) for j in range(16)]

    def _bcast(vec, j):
        return lax.gather(vec, idxs[j], dn, slice_sizes=(1,),
                          mode=lax.GatherScatterMode.PROMISE_IN_BOUNDS)

    num_blocks = D_FEAT // DB
    for db in range(num_blocks):
        s_w = [s_v[pl.ds((db * DB + j) * K, K)] for j in range(DB)]
        m_w = [m_v[pl.ds((db * DB + j) * K, K)] for j in range(DB)]
        half = (db * DB // 16) * 16
        off = db * DB - half

        @plsc.parallel_loop(0, CHUNK, 1, unroll=2)
        def _token_body(t, _db=db, _s=s_w, _m=m_w, _half=half, _off=off):
            xv = x_v[pl.ds(t * D_FEAT + _half, 16)]
            if _db == 0:
                acc_a = jnp.zeros((K,), jnp.float32)
            else:
                acc_a = out_v[pl.ds(t * K, K)]
            acc_b = jnp.zeros((K,), jnp.float32)
            for j in range(0, DB, 2):
                z0 = _bcast(xv, _off + j) * _s[j] - _m[j]
                acc_a = acc_a + z0 * z0
                z1 = _bcast(xv, _off + j + 1) * _s[j + 1] - _m[j + 1]
                acc_b = acc_b + z1 * z1
            acc = acc_a + acc_b
            if _db == num_blocks - 1:
                acc = c - 0.5 * acc
            out_v[pl.ds(t * K, K)] = acc

    pltpu.sync_copy(out_v, out_hbm.at[pl.ds(base * K, CHUNK * K)])


def _sc_log_probs(X_sc, means, log_scales):
    mesh = plsc.VectorSubcoreMesh(
        core_axis_name="c", subcore_axis_name="s", num_cores=1)
    run = pl.kernel(
        _sc_body,
        out_type=jax.ShapeDtypeStruct((T_SC * K,), jnp.float32),
        mesh=mesh,
        scratch_types=[
            pltpu.VMEM((CHUNK * D_FEAT,), jnp.float32),
            pltpu.VMEM((CHUNK * K,), jnp.float32),
            pltpu.VMEM((D_FEAT * K,), jnp.float32),
            pltpu.VMEM((D_FEAT * K,), jnp.float32),
        ],
    )
    out = run(X_sc.reshape(-1), means.T.reshape(-1), log_scales.T.reshape(-1))
    return out.reshape(T_SC, K)


def _tc_body(x_ref, mu_ref, ls_ref, o_ref):
    mu = mu_ref[...]
    ls = ls_ref[...]
    iv = jnp.exp(-2.0 * ls)
    w = jnp.concatenate([mu * iv, -0.5 * iv], axis=1)
    c = (jnp.sum(-0.5 * mu * mu * iv - ls, axis=1)
         - 0.5 * D_FEAT * LOG2PI)
    x = x_ref[...]
    xx = jnp.concatenate([x, x * x], axis=1)
    o_ref[...] = jax.lax.dot_general(
        xx, w, (((1,), (1,)), ((), ())),
        preferred_element_type=jnp.float32) + c[None, :]


def _tc_log_probs(X_tc, means, log_scales):
    t_tc = X_tc.shape[0]
    return pl.pallas_call(
        _tc_body,
        grid=(t_tc // TC_BLOCK,),
        in_specs=[
            pl.BlockSpec((TC_BLOCK, D_FEAT), lambda i: (i, 0)),
            pl.BlockSpec((K, D_FEAT), lambda i: (0, 0)),
            pl.BlockSpec((K, D_FEAT), lambda i: (0, 0)),
        ],
        out_specs=pl.BlockSpec((TC_BLOCK, K), lambda i: (i, 0)),
        out_shape=jax.ShapeDtypeStruct((t_tc, K), jnp.float32),
    )(X_tc, means, log_scales)


@jax.jit
def _emission_log_probs(X, means, log_scales):
    out_sc = _sc_log_probs(X[:T_SC], means, log_scales)
    out_tc = _tc_log_probs(X[T_SC:], means, log_scales)
    return jnp.concatenate([out_sc, out_tc], axis=0)


def kernel(X, cu_seqlens, means, log_scales, pi_logits, A_logits, D_logits):
    return _emission_log_probs(
        X.astype(jnp.float32),
        means.astype(jnp.float32),
        log_scales.astype(jnp.float32),
    )


# speedup vs baseline: 2.7383x; 1.4379x over previous
import math

import jax
import jax.numpy as jnp
from jax import lax
from jax.experimental import pallas as pl
from jax.experimental.pallas import tpu as pltpu
from jax.experimental.pallas import tpu_sc as plsc

T = 32768
D_FEAT = 32
K = 16
NW = 16
T_SC = 2048
CHUNK = T_SC // NW
DB = 16
TC_BLOCK = 4096
LOG2PI = math.log(2.0 * math.pi)


def _sc_body(x_hbm, mu_t_hbm, ls_t_hbm, out_hbm, x_v, out_v, s_v, m_v):
    wid = lax.axis_index("s")
    base = wid * CHUNK

    pltpu.sync_copy(x_hbm.at[pl.ds(base * D_FEAT, CHUNK * D_FEAT)], x_v)
    pltpu.sync_copy(mu_t_hbm, m_v)
    pltpu.sync_copy(ls_t_hbm, s_v)

    c = jnp.full((K,), -0.5 * D_FEAT * LOG2PI, jnp.float32)
    for d in range(D_FEAT):
        mu = m_v[pl.ds(d * K, K)]
        ls = s_v[pl.ds(d * K, K)]
        s = jnp.exp(-ls)
        c = c - ls
        s_v[pl.ds(d * K, K)] = s
        m_v[pl.ds(d * K, K)] = mu * s

    dn = lax.GatherDimensionNumbers(
        offset_dims=(), collapsed_slice_dims=(0,), start_index_map=(0,))
    idxs = [jnp.full((16, 1), j, jnp.int32) for j in range(16)]

    def _bcast(vec, j):
        return lax.gather(vec, idxs[j], dn, slice_sizes=(1,),
                          mode=lax.GatherScatterMode.PROMISE_IN_BOUNDS)

    num_blocks = D_FEAT // DB
    for db in range(num_blocks):
        s_w = [s_v[pl.ds((db * DB + j) * K, K)] for j in range(DB)]
        m_w = [m_v[pl.ds((db * DB + j) * K, K)] for j in range(DB)]
        half = (db * DB // 16) * 16
        off = db * DB - half

        @plsc.parallel_loop(0, CHUNK, 1, unroll=2)
        def _token_body(t, _db=db, _s=s_w, _m=m_w, _half=half, _off=off):
            xv = x_v[pl.ds(t * D_FEAT + _half, 16)]
            if _db == 0:
                acc_a = jnp.zeros((K,), jnp.float32)
            else:
                acc_a = out_v[pl.ds(t * K, K)]
            acc_b = jnp.zeros((K,), jnp.float32)
            for j in range(0, DB, 2):
                z0 = _bcast(xv, _off + j) * _s[j] - _m[j]
                acc_a = acc_a + z0 * z0
                z1 = _bcast(xv, _off + j + 1) * _s[j + 1] - _m[j + 1]
                acc_b = acc_b + z1 * z1
            acc = acc_a + acc_b
            if _db == num_blocks - 1:
                acc = c - 0.5 * acc
            out_v[pl.ds(t * K, K)] = acc

    pltpu.sync_copy(out_v, out_hbm.at[pl.ds(base * K, CHUNK * K)])


def _sc_log_probs(X_sc, means, log_scales):
    mesh = plsc.VectorSubcoreMesh(
        core_axis_name="c", subcore_axis_name="s", num_cores=1)
    run = pl.kernel(
        _sc_body,
        out_type=jax.ShapeDtypeStruct((T_SC * K,), jnp.float32),
        mesh=mesh,
        scratch_types=[
            pltpu.VMEM((CHUNK * D_FEAT,), jnp.float32),
            pltpu.VMEM((CHUNK * K,), jnp.float32),
            pltpu.VMEM((D_FEAT * K,), jnp.float32),
            pltpu.VMEM((D_FEAT * K,), jnp.float32),
        ],
    )
    out = run(X_sc.reshape(-1), means.T.reshape(-1), log_scales.T.reshape(-1))
    return out.reshape(T_SC, K)


def _tc_body(x_ref, mu_ref, ls_ref, o_ref):
    mu = mu_ref[...]
    ls = ls_ref[...]
    iv = jnp.exp(-2.0 * ls)
    w = jnp.concatenate([mu * iv, -0.5 * iv], axis=1)
    c = (jnp.sum(-0.5 * mu * mu * iv - ls, axis=1)
         - 0.5 * D_FEAT * LOG2PI)
    x = x_ref[...]
    xx = jnp.concatenate([x, x * x], axis=1)
    o_ref[...] = jax.lax.dot_general(
        xx, w, (((1,), (1,)), ((), ())),
        preferred_element_type=jnp.float32) + c[None, :]


def _tc_log_probs(X_tc, means, log_scales):
    t_tc = X_tc.shape[0]
    return pl.pallas_call(
        _tc_body,
        grid=(t_tc // TC_BLOCK,),
        in_specs=[
            pl.BlockSpec((TC_BLOCK, D_FEAT), lambda i: (i, 0)),
            pl.BlockSpec((K, D_FEAT), lambda i: (0, 0)),
            pl.BlockSpec((K, D_FEAT), lambda i: (0, 0)),
        ],
        out_specs=pl.BlockSpec((TC_BLOCK, K), lambda i: (i, 0)),
        out_shape=jax.ShapeDtypeStruct((t_tc, K), jnp.float32),
    )(X_tc, means, log_scales)


@jax.jit
def _emission_log_probs(X, means, log_scales):
    return _tc_log_probs(X, means, log_scales)


def kernel(X, cu_seqlens, means, log_scales, pi_logits, A_logits, D_logits):
    return _emission_log_probs(
        X.astype(jnp.float32),
        means.astype(jnp.float32),
        log_scales.astype(jnp.float32),
    )


# speedup vs baseline: 2.7482x; 1.0036x over previous
import math

import jax
import jax.numpy as jnp
from jax import lax
from jax.experimental import pallas as pl
from jax.experimental.pallas import tpu as pltpu
from jax.experimental.pallas import tpu_sc as plsc

T = 32768
D_FEAT = 32
K = 16
NW = 16
T_SC = 2048
CHUNK = T_SC // NW
DB = 16
TC_BLOCK = 4096
LOG2PI = math.log(2.0 * math.pi)


def _sc_body(x_hbm, mu_t_hbm, ls_t_hbm, out_hbm, x_v, out_v, s_v, m_v):
    wid = lax.axis_index("s")
    base = wid * CHUNK

    pltpu.sync_copy(x_hbm.at[pl.ds(base * D_FEAT, CHUNK * D_FEAT)], x_v)
    pltpu.sync_copy(mu_t_hbm, m_v)
    pltpu.sync_copy(ls_t_hbm, s_v)

    c = jnp.full((K,), -0.5 * D_FEAT * LOG2PI, jnp.float32)
    for d in range(D_FEAT):
        mu = m_v[pl.ds(d * K, K)]
        ls = s_v[pl.ds(d * K, K)]
        s = jnp.exp(-ls)
        c = c - ls
        s_v[pl.ds(d * K, K)] = s
        m_v[pl.ds(d * K, K)] = mu * s

    dn = lax.GatherDimensionNumbers(
        offset_dims=(), collapsed_slice_dims=(0,), start_index_map=(0,))
    idxs = [jnp.full((16, 1), j, jnp.int32) for j in range(16)]

    def _bcast(vec, j):
        return lax.gather(vec, idxs[j], dn, slice_sizes=(1,),
                          mode=lax.GatherScatterMode.PROMISE_IN_BOUNDS)

    num_blocks = D_FEAT // DB
    for db in range(num_blocks):
        s_w = [s_v[pl.ds((db * DB + j) * K, K)] for j in range(DB)]
        m_w = [m_v[pl.ds((db * DB + j) * K, K)] for j in range(DB)]
        half = (db * DB // 16) * 16
        off = db * DB - half

        @plsc.parallel_loop(0, CHUNK, 1, unroll=2)
        def _token_body(t, _db=db, _s=s_w, _m=m_w, _half=half, _off=off):
            xv = x_v[pl.ds(t * D_FEAT + _half, 16)]
            if _db == 0:
                acc_a = jnp.zeros((K,), jnp.float32)
            else:
                acc_a = out_v[pl.ds(t * K, K)]
            acc_b = jnp.zeros((K,), jnp.float32)
            for j in range(0, DB, 2):
                z0 = _bcast(xv, _off + j) * _s[j] - _m[j]
                acc_a = acc_a + z0 * z0
                z1 = _bcast(xv, _off + j + 1) * _s[j + 1] - _m[j + 1]
                acc_b = acc_b + z1 * z1
            acc = acc_a + acc_b
            if _db == num_blocks - 1:
                acc = c - 0.5 * acc
            out_v[pl.ds(t * K, K)] = acc

    pltpu.sync_copy(out_v, out_hbm.at[pl.ds(base * K, CHUNK * K)])


def _sc_log_probs(X_sc, means, log_scales):
    mesh = plsc.VectorSubcoreMesh(
        core_axis_name="c", subcore_axis_name="s", num_cores=1)
    run = pl.kernel(
        _sc_body,
        out_type=jax.ShapeDtypeStruct((T_SC * K,), jnp.float32),
        mesh=mesh,
        scratch_types=[
            pltpu.VMEM((CHUNK * D_FEAT,), jnp.float32),
            pltpu.VMEM((CHUNK * K,), jnp.float32),
            pltpu.VMEM((D_FEAT * K,), jnp.float32),
            pltpu.VMEM((D_FEAT * K,), jnp.float32),
        ],
    )
    out = run(X_sc.reshape(-1), means.T.reshape(-1), log_scales.T.reshape(-1))
    return out.reshape(T_SC, K)


def _tc_body(x_ref, mu_ref, ls_ref, o_ref):
    mu = mu_ref[...]
    ls = ls_ref[...]
    iv = jnp.exp(-2.0 * ls)
    c = (jnp.sum(-0.5 * mu * mu * iv - ls, axis=1)
         - 0.5 * D_FEAT * LOG2PI)
    x = x_ref[...]
    w1 = mu * iv
    w2 = -0.5 * iv
    o_ref[...] = (
        jax.lax.dot_general(x, w1, (((1,), (1,)), ((), ())),
                            preferred_element_type=jnp.float32)
        + jax.lax.dot_general(x * x, w2, (((1,), (1,)), ((), ())),
                              preferred_element_type=jnp.float32)
        + c[None, :])


def _tc_log_probs(X_tc, means, log_scales):
    t_tc = X_tc.shape[0]
    return pl.pallas_call(
        _tc_body,
        grid=(t_tc // TC_BLOCK,),
        in_specs=[
            pl.BlockSpec((TC_BLOCK, D_FEAT), lambda i: (i, 0)),
            pl.BlockSpec((K, D_FEAT), lambda i: (0, 0)),
            pl.BlockSpec((K, D_FEAT), lambda i: (0, 0)),
        ],
        out_specs=pl.BlockSpec((TC_BLOCK, K), lambda i: (i, 0)),
        out_shape=jax.ShapeDtypeStruct((t_tc, K), jnp.float32),
    )(X_tc, means, log_scales)


@jax.jit
def _emission_log_probs(X, means, log_scales):
    return _tc_log_probs(X, means, log_scales)


def kernel(X, cu_seqlens, means, log_scales, pi_logits, A_logits, D_logits):
    return _emission_log_probs(
        X.astype(jnp.float32),
        means.astype(jnp.float32),
        log_scales.astype(jnp.float32),
    )
